# Initial kernel scaffold; baseline (speedup 1.0000x reference)
import jax
import jax.numpy as jnp
from jax.experimental import pallas as pl


def kernel(x, edge_index, edge_attr, batch, params):
    raise NotImplementedError("write your pallas kernel here")


# speedup vs baseline: 1.3218x; 1.3218x over previous
import jax
import jax.numpy as jnp
from jax.experimental import pallas as pl

_PH = jax.lax.Precision.HIGHEST
_N = 10000
_G = 64


def _bf16r(x):
    return x.astype(jnp.bfloat16).astype(jnp.float32)


def _lrelu(x, s):
    return jnp.where(x >= 0, x, s * x)


def _bnf(p, x):
    mu = x.mean(axis=0)
    var = x.var(axis=0)
    return p["gamma"] * (x - mu) / jnp.sqrt(var + 1e-5) + p["beta"]


def _gru(p, inp, h):
    gi = inp @ p["W_ih"] + p["b_ih"]
    gh = h @ p["W_hh"] + p["b_hh"]
    i_r, i_z, i_n = jnp.split(gi, 3, axis=-1)
    h_r, h_z, h_n = jnp.split(gh, 3, axis=-1)
    r = jax.nn.sigmoid(i_r + h_r)
    z = jax.nn.sigmoid(i_z + h_z)
    n = jnp.tanh(i_n + r * h_n)
    return (1.0 - z) * n + z * h


def _head(p, ex, ee, src, dst, batch):
    h0 = _lrelu(ex @ p["lin1"]["W"] + p["lin1"]["b"], 0.01)
    g = p["gate"]
    P = h0 @ g["lin1_W"][:128] + g["lin1_b"]
    Q = ee @ g["lin1_W"][128:]
    r = (h0 * g["att_r"]).sum(-1)
    he = _lrelu(P[src] + Q, 0.01)
    a = _lrelu((he * g["att_l"]).sum(-1) + r[dst], 0.01)
    e = jnp.exp(a)
    d = jax.ops.segment_sum(e, dst, num_segments=_N)
    u = jax.ops.segment_sum(e[:, None] * _bf16r(he), dst, num_segments=_N)
    acc = u / (d + 1e-16)[:, None]
    m = jax.nn.elu(jnp.dot(acc, _bf16r(g["lin2_W"]), precision=_PH) + g["bias"])
    hx = _gru(p["gru0"], m, h0)

    gp = p["atom"][0]
    xp = hx @ gp["W"]
    s = (xp * gp["att_src"]).sum(-1)
    t = (xp * gp["att_dst"]).sum(-1)
    a = _lrelu(s[src] + t[dst], 0.2)
    e = jnp.exp(a)
    d = jax.ops.segment_sum(e, dst, num_segments=_N)
    u = jax.ops.segment_sum(e[:, None] * xp[src], dst, num_segments=_N)
    m = jax.nn.elu(u / (d + 1e-16)[:, None] + gp["bias"])
    hx = _gru(p["agru"][0], m, hx)

    out = jax.nn.relu(jax.ops.segment_sum(hx, batch, num_segments=_G))
    mp = p["mol"]
    xp = hx @ mp["W"]
    s = (xp * mp["att_src"]).sum(-1)
    for _ in range(2):
        op = out @ mp["W"]
        tg = (op * mp["att_dst"]).sum(-1)
        a = _lrelu(s + tg[batch], 0.2)
        e = jnp.exp(a)
        d = jax.ops.segment_sum(e, batch, num_segments=_G)
        u = jax.ops.segment_sum(e[:, None] * xp, batch, num_segments=_G)
        m = jax.nn.elu(u / (d + 1e-16)[:, None] + mp["bias"])
        out = jax.nn.relu(_gru(p["mgru"], m, out))
    return out @ p["lin2"]["W"] + p["lin2"]["b"]


def kernel(x, edge_index, edge_attr, batch, params):
    src = edge_index[0]
    dst = edge_index[1]
    ex = _lrelu(_bnf(params["node_bn"], x @ params["node_lin"]["W"] + params["node_lin"]["b"]), 0.01)
    ee = _lrelu(_bnf(params["edge_bn"], edge_attr @ params["edge_lin"]["W"] + params["edge_lin"]["b"]), 0.01)
    heads = [_head(hp, ex, ee, src, dst, batch) for hp in params["heads"]]
    cat = jnp.concatenate(heads, axis=-1)
    return jax.nn.relu(_bnf(params["att_bn"], cat @ params["att_lin"]["W"] + params["att_lin"]["b"]))


# speedup vs baseline: 5.7209x; 4.3281x over previous
import functools

import jax
import jax.numpy as jnp
from jax import lax
from jax.experimental import pallas as pl
from jax.experimental.pallas import tpu as pltpu
from jax.experimental.pallas import tpu_sc as plsc

_PH = jax.lax.Precision.HIGHEST
_N = 10000
_E = 320000
_G = 64
_H = 128
_C = 80
_EPT = _E // 32
_NCH = _EPT // _C
_NP = 10112
_RPT = _NP // 16


def _bf16r(x):
    return x.astype(jnp.bfloat16).astype(jnp.float32)


def _lrelu(x, s):
    return jnp.where(x >= 0, x, s * x)


def _zero_ref(ref, n):
    z = jnp.zeros((16,), jnp.float32)
    if len(ref.shape) == 1:
        def body(i, _):
            ref[pl.ds(i * 16, 16)] = z
            return 0
    else:
        w = ref.shape[1] // 16

        def body(i, _):
            ref[i // w, pl.ds((i % w) * 16, 16)] = z
            return 0
    lax.fori_loop(0, n, body, 0)


def _edge_kernel_body(has_q, slope,
                      src_h, dst_h, tabp_h, sca_h, scb_h, q_h, attl_h,
                      u_h, d_h, *scratch):
    if has_q:
        srcv, dstv, pg, qv, sbuf, wbuf, scb_v, attl_v, dtile, ush, sem = scratch
        sca_v = None
    else:
        srcv, dstv, pg, sca_v, scb_v, wbuf, dtile, ush, sem = scratch
        qv = sbuf = attl_v = None
    c = lax.axis_index("c")
    s = lax.axis_index("s")
    wid = c * 16 + s

    if not has_q:
        pltpu.sync_copy(sca_h, sca_v)
    pltpu.sync_copy(scb_h, scb_v)
    if has_q:
        pltpu.sync_copy(attl_h, attl_v)

    _zero_ref(pg, _C * 8)
    _zero_ref(dtile, _N // 16)
    for j in range(7):
        pltpu.sync_copy(pg, ush.at[pl.ds(s * _RPT + j * _C, _C)])
    pltpu.sync_copy(pg.at[pl.ds(0, 72)], ush.at[pl.ds(s * _RPT + 560, 72)])
    plsc.subcore_barrier()

    def chunk(g, _):
        base = wid * _EPT + g * _C
        pltpu.sync_copy(src_h.at[pl.ds(base, _C)], srcv)
        pltpu.sync_copy(dst_h.at[pl.ds(base, _C)], dstv)
        if has_q:
            pltpu.sync_copy(q_h.at[pl.ds(base, _C)], qv)
        pltpu.async_copy(tabp_h.at[srcv], pg, sem).wait()

        if has_q:
            def edge(i, _):
                acc = jnp.zeros((16,), jnp.float32)
                for k in range(8):
                    p = pg[i, pl.ds(16 * k, 16)]
                    q = qv[i, pl.ds(16 * k, 16)]
                    al = attl_v[pl.ds(16 * k, 16)]
                    he = p + q
                    he = jnp.where(he >= 0, he, 0.01 * he)
                    acc = acc + he * al
                    yi = plsc.bitcast(he, jnp.int32)
                    yi = (yi + 32767 + ((yi >> 16) & 1)) & (-65536)
                    pg[i, pl.ds(16 * k, 16)] = plsc.bitcast(yi, jnp.float32)
                tot = plsc.cumsum(acc)
                plsc.store_scatter(sbuf, [jnp.full((16,), i, jnp.int32)],
                                   tot, mask=lax.iota(jnp.int32, 16) >= 15)
                return 0
            lax.fori_loop(0, _C, edge, 0)

        for j in range(_C // 16):
            dst16 = dstv[pl.ds(16 * j, 16)]
            if has_q:
                s16 = sbuf[pl.ds(16 * j, 16)]
                rg = plsc.load_gather(scb_v, [dst16])
                a = s16 + rg
            else:
                src16 = srcv[pl.ds(16 * j, 16)]
                a = plsc.load_gather(sca_v, [src16]) + plsc.load_gather(scb_v, [dst16])
            a = jnp.where(a >= 0, a, slope * a)
            w = jnp.exp(a)
            wbuf[pl.ds(16 * j, 16)] = w
            plsc.addupdate_scatter(dtile, [dst16], w)

        def scale(i, _):
            wspl = plsc.load_gather(wbuf, [jnp.full((16,), i, jnp.int32)])
            for k in range(8):
                pg[i, pl.ds(16 * k, 16)] = pg[i, pl.ds(16 * k, 16)] * wspl
            return 0
        lax.fori_loop(0, _C, scale, 0)

        pltpu.sync_copy(pg, ush.at[dstv], add=True)
        return 0

    lax.fori_loop(0, _NCH, chunk, 0)
    plsc.subcore_barrier()

    for j in range(7):
        r0 = s * _RPT + j * _C
        pltpu.sync_copy(ush.at[pl.ds(r0, _C)], u_h.at[c, pl.ds(r0, _C)])
    r0 = s * _RPT + 560
    pltpu.sync_copy(ush.at[pl.ds(r0, 72)], u_h.at[c, pl.ds(r0, 72)])
    pltpu.sync_copy(dtile, d_h.at[wid, 0])


def _make_edge_pass(has_q, slope):
    mesh = plsc.VectorSubcoreMesh(core_axis_name="c", subcore_axis_name="s")
    body = functools.partial(_edge_kernel_body, has_q, slope)
    return pl.kernel(
        body,
        out_type=(jax.ShapeDtypeStruct((2, _NP, _H), jnp.float32),
                  jax.ShapeDtypeStruct((32, 1, _N), jnp.float32)),
        mesh=mesh,
        compiler_params=pltpu.CompilerParams(needs_layout_passes=False),
        scratch_types=(
            [pltpu.VMEM((_C,), jnp.int32),
             pltpu.VMEM((_C,), jnp.int32)]
            + ([pltpu.VMEM((_C, _H), jnp.float32),
                pltpu.VMEM((_C, _H), jnp.float32),
                pltpu.VMEM((_C,), jnp.float32),
                pltpu.VMEM((_C,), jnp.float32),
                pltpu.VMEM((_N,), jnp.float32),
                pltpu.VMEM((_H,), jnp.float32)]
               if has_q else
               [pltpu.VMEM((_C, _H), jnp.float32),
                pltpu.VMEM((_N,), jnp.float32),
                pltpu.VMEM((_N,), jnp.float32),
                pltpu.VMEM((_C,), jnp.float32)])
            + [pltpu.VMEM((_N,), jnp.float32),
               pltpu.VMEM_SHARED((_NP, _H), jnp.float32),
               pltpu.SemaphoreType.DMA]
        ),
    )


_gate_pass = _make_edge_pass(True, 0.01)
_gat_pass = _make_edge_pass(False, 0.2)


def _gate_edge_sc(P, r, Q, src, dst, att_l):
    dummy = jnp.zeros((_N,), jnp.float32)
    U, D = _gate_pass(src, dst, P, dummy, r, Q, att_l)
    return (U[0] + U[1])[:_N], D.sum(0)[0]


def _gat_edge_sc(XP, s, t, src, dst):
    dummyq = jnp.zeros((1, _H), jnp.float32)
    dummyl = jnp.zeros((_H,), jnp.float32)
    U, D = _gat_pass(src, dst, XP, s, t, dummyq, dummyl)
    return (U[0] + U[1])[:_N], D.sum(0)[0]


def _bnf(p, x):
    mu = x.mean(axis=0)
    var = x.var(axis=0)
    return p["gamma"] * (x - mu) / jnp.sqrt(var + 1e-5) + p["beta"]


def _gru(p, inp, h):
    gi = inp @ p["W_ih"] + p["b_ih"]
    gh = h @ p["W_hh"] + p["b_hh"]
    i_r, i_z, i_n = jnp.split(gi, 3, axis=-1)
    h_r, h_z, h_n = jnp.split(gh, 3, axis=-1)
    r = jax.nn.sigmoid(i_r + h_r)
    z = jax.nn.sigmoid(i_z + h_z)
    n = jnp.tanh(i_n + r * h_n)
    return (1.0 - z) * n + z * h


def _head(p, ex, ee, src, dst, batch):
    h0 = _lrelu(ex @ p["lin1"]["W"] + p["lin1"]["b"], 0.01)
    g = p["gate"]
    P = h0 @ g["lin1_W"][:128] + g["lin1_b"]
    Q = ee @ g["lin1_W"][128:]
    r = (h0 * g["att_r"]).sum(-1)
    u, d = _gate_edge_sc(P, r, Q, src, dst, g["att_l"])
    acc = u / (d + 1e-16)[:, None]
    m = jax.nn.elu(jnp.dot(acc, _bf16r(g["lin2_W"]), precision=_PH) + g["bias"])
    hx = _gru(p["gru0"], m, h0)

    gp = p["atom"][0]
    xp = hx @ gp["W"]
    s = (xp * gp["att_src"]).sum(-1)
    t = (xp * gp["att_dst"]).sum(-1)
    u, d = _gat_edge_sc(xp, s, t, src, dst)
    m = jax.nn.elu(u / (d + 1e-16)[:, None] + gp["bias"])
    hx = _gru(p["agru"][0], m, hx)

    out = jax.nn.relu(jax.ops.segment_sum(hx, batch, num_segments=_G))
    mp = p["mol"]
    xp = hx @ mp["W"]
    s = (xp * mp["att_src"]).sum(-1)
    for _ in range(2):
        op = out @ mp["W"]
        tg = (op * mp["att_dst"]).sum(-1)
        a = _lrelu(s + tg[batch], 0.2)
        e = jnp.exp(a)
        d = jax.ops.segment_sum(e, batch, num_segments=_G)
        u = jax.ops.segment_sum(e[:, None] * xp, batch, num_segments=_G)
        m = jax.nn.elu(u / (d + 1e-16)[:, None] + mp["bias"])
        out = jax.nn.relu(_gru(p["mgru"], m, out))
    return out @ p["lin2"]["W"] + p["lin2"]["b"]


def kernel(x, edge_index, edge_attr, batch, params):
    src = edge_index[0]
    dst = edge_index[1]
    ex = _lrelu(_bnf(params["node_bn"], x @ params["node_lin"]["W"] + params["node_lin"]["b"]), 0.01)
    ee = _lrelu(_bnf(params["edge_bn"], edge_attr @ params["edge_lin"]["W"] + params["edge_lin"]["b"]), 0.01)
    heads = [_head(hp, ex, ee, src, dst, batch) for hp in params["heads"]]
    cat = jnp.concatenate(heads, axis=-1)
    return jax.nn.relu(_bnf(params["att_bn"], cat @ params["att_lin"]["W"] + params["att_lin"]["b"]))


# speedup vs baseline: 8.4065x; 1.4694x over previous
import functools

import jax
import jax.numpy as jnp
from jax import lax
from jax.experimental import pallas as pl
from jax.experimental.pallas import tpu as pltpu
from jax.experimental.pallas import tpu_sc as plsc

_PH = jax.lax.Precision.HIGHEST
_N = 10000
_E = 320000
_G = 64
_H = 128
_C = 80
_EPT = _E // 32
_NCH = _EPT // _C
_NP = 10112
_RPT = _NP // 16


def _bf16r(x):
    return x.astype(jnp.bfloat16).astype(jnp.float32)


def _lrelu(x, s):
    return jnp.where(x >= 0, x, s * x)


def _zero_ref(ref, n):
    z = jnp.zeros((16,), jnp.float32)
    if len(ref.shape) == 1:
        def body(i, _):
            ref[pl.ds(i * 16, 16)] = z
            return 0
    else:
        w = ref.shape[1] // 16

        def body(i, _):
            ref[i // w, pl.ds((i % w) * 16, 16)] = z
            return 0
    lax.fori_loop(0, n, body, 0)


def _edge_kernel_body(has_q, slope,
                      src_h, dst_h, tabp_h, sca_h, scb_h, q_h, attl_h,
                      u_h, d_h, *scratch):
    if has_q:
        srcv, dstv, pg, qv, sbuf, wbuf, scb_v, attl_v, dtile, ush, sem = scratch
        sca_v = None
    else:
        srcv, dstv, pg, sca_v, scb_v, wbuf, dtile, ush, sem = scratch
        qv = sbuf = attl_v = None
    c = lax.axis_index("c")
    s = lax.axis_index("s")
    wid = c * 16 + s

    if not has_q:
        pltpu.sync_copy(sca_h, sca_v)
    pltpu.sync_copy(scb_h, scb_v)
    if has_q:
        pltpu.sync_copy(attl_h, attl_v)

    _zero_ref(pg, _C * 8)
    _zero_ref(dtile, _N // 16)
    for j in range(7):
        pltpu.sync_copy(pg, ush.at[pl.ds(s * _RPT + j * _C, _C)])
    pltpu.sync_copy(pg.at[pl.ds(0, 72)], ush.at[pl.ds(s * _RPT + 560, 72)])
    plsc.subcore_barrier()

    def chunk(g, _):
        base = wid * _EPT + g * _C
        pltpu.sync_copy(src_h.at[pl.ds(base, _C)], srcv)
        pltpu.sync_copy(dst_h.at[pl.ds(base, _C)], dstv)
        if has_q:
            pltpu.sync_copy(q_h.at[pl.ds(base, _C)], qv)
        pltpu.async_copy(tabp_h.at[srcv], pg, sem).wait()

        if has_q:
            als = [attl_v[pl.ds(16 * k, 16)] for k in range(8)]
            last = lax.iota(jnp.int32, 16) >= 15

            @plsc.parallel_loop(0, _C, unroll=4)
            def edge(i):
                acc = jnp.zeros((16,), jnp.float32)
                for k in range(8):
                    p = pg[i, pl.ds(16 * k, 16)]
                    q = qv[i, pl.ds(16 * k, 16)]
                    he = p + q
                    he = jnp.where(he >= 0, he, 0.01 * he)
                    acc = acc + he * als[k]
                    yi = plsc.bitcast(he, jnp.int32)
                    yi = (yi + 32767 + ((yi >> 16) & 1)) & (-65536)
                    pg[i, pl.ds(16 * k, 16)] = plsc.bitcast(yi, jnp.float32)
                tot = plsc.cumsum(acc)
                plsc.store_scatter(sbuf, [jnp.full((16,), i, jnp.int32)],
                                   tot, mask=last)

        for j in range(_C // 16):
            dst16 = dstv[pl.ds(16 * j, 16)]
            if has_q:
                s16 = sbuf[pl.ds(16 * j, 16)]
                rg = plsc.load_gather(scb_v, [dst16])
                a = s16 + rg
            else:
                src16 = srcv[pl.ds(16 * j, 16)]
                a = plsc.load_gather(sca_v, [src16]) + plsc.load_gather(scb_v, [dst16])
            a = jnp.where(a >= 0, a, slope * a)
            w = jnp.exp(a)
            wbuf[pl.ds(16 * j, 16)] = w
            plsc.addupdate_scatter(dtile, [dst16], w)

        @plsc.parallel_loop(0, _C, unroll=4)
        def scale(i):
            wspl = plsc.load_gather(wbuf, [jnp.full((16,), i, jnp.int32)])
            for k in range(8):
                pg[i, pl.ds(16 * k, 16)] = pg[i, pl.ds(16 * k, 16)] * wspl

        pltpu.sync_copy(pg, ush.at[dstv], add=True)
        return 0

    lax.fori_loop(0, _NCH, chunk, 0)
    plsc.subcore_barrier()

    for j in range(7):
        r0 = s * _RPT + j * _C
        pltpu.sync_copy(ush.at[pl.ds(r0, _C)], u_h.at[c, pl.ds(r0, _C)])
    r0 = s * _RPT + 560
    pltpu.sync_copy(ush.at[pl.ds(r0, 72)], u_h.at[c, pl.ds(r0, 72)])
    pltpu.sync_copy(dtile, d_h.at[wid, 0])


def _make_edge_pass(has_q, slope):
    mesh = plsc.VectorSubcoreMesh(core_axis_name="c", subcore_axis_name="s")
    body = functools.partial(_edge_kernel_body, has_q, slope)
    return pl.kernel(
        body,
        out_type=(jax.ShapeDtypeStruct((2, _NP, _H), jnp.float32),
                  jax.ShapeDtypeStruct((32, 1, _N), jnp.float32)),
        mesh=mesh,
        compiler_params=pltpu.CompilerParams(needs_layout_passes=False),
        scratch_types=(
            [pltpu.VMEM((_C,), jnp.int32),
             pltpu.VMEM((_C,), jnp.int32)]
            + ([pltpu.VMEM((_C, _H), jnp.float32),
                pltpu.VMEM((_C, _H), jnp.float32),
                pltpu.VMEM((_C,), jnp.float32),
                pltpu.VMEM((_C,), jnp.float32),
                pltpu.VMEM((_N,), jnp.float32),
                pltpu.VMEM((_H,), jnp.float32)]
               if has_q else
               [pltpu.VMEM((_C, _H), jnp.float32),
                pltpu.VMEM((_N,), jnp.float32),
                pltpu.VMEM((_N,), jnp.float32),
                pltpu.VMEM((_C,), jnp.float32)])
            + [pltpu.VMEM((_N,), jnp.float32),
               pltpu.VMEM_SHARED((_NP, _H), jnp.float32),
               pltpu.SemaphoreType.DMA]
        ),
    )


_gate_pass = _make_edge_pass(True, 0.01)
_gat_pass = _make_edge_pass(False, 0.2)


def _gate_edge_sc(P, r, Q, src, dst, att_l):
    dummy = jnp.zeros((_N,), jnp.float32)
    U, D = _gate_pass(src, dst, P, dummy, r, Q, att_l)
    return (U[0] + U[1])[:_N], D.sum(0)[0]


def _gat_edge_sc(XP, s, t, src, dst):
    dummyq = jnp.zeros((1, _H), jnp.float32)
    dummyl = jnp.zeros((_H,), jnp.float32)
    U, D = _gat_pass(src, dst, XP, s, t, dummyq, dummyl)
    return (U[0] + U[1])[:_N], D.sum(0)[0]


def _bnf(p, x):
    mu = x.mean(axis=0)
    var = x.var(axis=0)
    return p["gamma"] * (x - mu) / jnp.sqrt(var + 1e-5) + p["beta"]


def _gru(p, inp, h):
    gi = inp @ p["W_ih"] + p["b_ih"]
    gh = h @ p["W_hh"] + p["b_hh"]
    i_r, i_z, i_n = jnp.split(gi, 3, axis=-1)
    h_r, h_z, h_n = jnp.split(gh, 3, axis=-1)
    r = jax.nn.sigmoid(i_r + h_r)
    z = jax.nn.sigmoid(i_z + h_z)
    n = jnp.tanh(i_n + r * h_n)
    return (1.0 - z) * n + z * h


def _head(p, ex, ee, src, dst, batch):
    h0 = _lrelu(ex @ p["lin1"]["W"] + p["lin1"]["b"], 0.01)
    g = p["gate"]
    P = h0 @ g["lin1_W"][:128] + g["lin1_b"]
    Q = ee @ g["lin1_W"][128:]
    r = (h0 * g["att_r"]).sum(-1)
    u, d = _gate_edge_sc(P, r, Q, src, dst, g["att_l"])
    acc = u / (d + 1e-16)[:, None]
    m = jax.nn.elu(jnp.dot(acc, _bf16r(g["lin2_W"]), precision=_PH) + g["bias"])
    hx = _gru(p["gru0"], m, h0)

    gp = p["atom"][0]
    xp = hx @ gp["W"]
    s = (xp * gp["att_src"]).sum(-1)
    t = (xp * gp["att_dst"]).sum(-1)
    u, d = _gat_edge_sc(xp, s, t, src, dst)
    m = jax.nn.elu(u / (d + 1e-16)[:, None] + gp["bias"])
    hx = _gru(p["agru"][0], m, hx)

    out = jax.nn.relu(jax.ops.segment_sum(hx, batch, num_segments=_G))
    mp = p["mol"]
    xp = hx @ mp["W"]
    s = (xp * mp["att_src"]).sum(-1)
    for _ in range(2):
        op = out @ mp["W"]
        tg = (op * mp["att_dst"]).sum(-1)
        a = _lrelu(s + tg[batch], 0.2)
        e = jnp.exp(a)
        d = jax.ops.segment_sum(e, batch, num_segments=_G)
        u = jax.ops.segment_sum(e[:, None] * xp, batch, num_segments=_G)
        m = jax.nn.elu(u / (d + 1e-16)[:, None] + mp["bias"])
        out = jax.nn.relu(_gru(p["mgru"], m, out))
    return out @ p["lin2"]["W"] + p["lin2"]["b"]


def kernel(x, edge_index, edge_attr, batch, params):
    src = edge_index[0]
    dst = edge_index[1]
    ex = _lrelu(_bnf(params["node_bn"], x @ params["node_lin"]["W"] + params["node_lin"]["b"]), 0.01)
    ee = _lrelu(_bnf(params["edge_bn"], edge_attr @ params["edge_lin"]["W"] + params["edge_lin"]["b"]), 0.01)
    heads = [_head(hp, ex, ee, src, dst, batch) for hp in params["heads"]]
    cat = jnp.concatenate(heads, axis=-1)
    return jax.nn.relu(_bnf(params["att_bn"], cat @ params["att_lin"]["W"] + params["att_lin"]["b"]))
